# baseline (speedup 1.0000x reference)
import functools

import jax
import jax.numpy as jnp
from jax import lax
from jax.experimental import pallas as pl
from jax.experimental.pallas import tpu as pltpu
from jax.experimental.pallas import tpu_sc as plsc

B, L, D = 4096, 200, 64
N = B * L


@functools.cache
def _build(n_rows, d):
    info = plsc.get_sparse_core_info()
    nw = info.num_cores * info.num_subcores
    per_w = n_rows // nw
    chunk = 512
    n_chunk = per_w // chunk
    mesh = plsc.VectorSubcoreMesh(core_axis_name="c", subcore_axis_name="s")

    @functools.partial(
        pl.kernel,
        mesh=mesh,
        out_type=jax.ShapeDtypeStruct((n_rows, d), jnp.float32),
        scratch_types=[
            pltpu.VMEM((chunk,), jnp.int32),
            pltpu.VMEM((chunk, d), jnp.float32),
            pltpu.SemaphoreType.DMA,
        ],
        compiler_params=pltpu.CompilerParams(use_tc_tiling_on_sc=False),
    )
    def gather_kernel(table_hbm, idx_hbm, out_hbm, idx_v, rows_v, sem):
        wid = lax.axis_index("s") * info.num_cores + lax.axis_index("c")
        base = wid * per_w

        def body(i, carry):
            off = base + i * chunk
            pltpu.sync_copy(idx_hbm.at[pl.ds(off, chunk)], idx_v)
            pltpu.async_copy(table_hbm.at[idx_v], rows_v, sem).wait()
            pltpu.sync_copy(rows_v, out_hbm.at[pl.ds(off, chunk)])
            return carry

        lax.fori_loop(0, n_chunk, body, 0)

    return gather_kernel


def kernel(batch, table):
    idx = batch.reshape(N).astype(jnp.int32)
    out = _build(N, D)(table, idx)
    return out.reshape(B, L, D)


# speedup vs baseline: 1.0433x; 1.0433x over previous
import functools

import jax
import jax.numpy as jnp
from jax import lax
from jax.experimental import pallas as pl
from jax.experimental.pallas import tpu as pltpu
from jax.experimental.pallas import tpu_sc as plsc

B, L, D = 4096, 200, 64
N = B * L
CHUNK = 512


@functools.cache
def _build(n_rows, d):
    info = plsc.get_sparse_core_info()
    nw = info.num_cores * info.num_subcores
    per_w = n_rows // nw
    n_chunk = per_w // CHUNK
    n_groups = n_chunk // 2
    mesh = plsc.VectorSubcoreMesh(core_axis_name="c", subcore_axis_name="s")

    @functools.partial(
        pl.kernel,
        mesh=mesh,
        out_type=jax.ShapeDtypeStruct((n_rows, d), jnp.float32),
        scratch_types=[
            pltpu.VMEM((per_w,), jnp.int32),
            pltpu.VMEM((CHUNK, d), jnp.float32),
            pltpu.VMEM((CHUNK, d), jnp.float32),
            pltpu.SemaphoreType.DMA,
            pltpu.SemaphoreType.DMA,
            pltpu.SemaphoreType.DMA,
            pltpu.SemaphoreType.DMA,
        ],
        compiler_params=pltpu.CompilerParams(use_tc_tiling_on_sc=False),
    )
    def gather_kernel(table_hbm, idx_hbm, out_hbm, idx_all, rows0, rows1,
                      gsem0, gsem1, osem0, osem1):
        wid = lax.axis_index("s") * info.num_cores + lax.axis_index("c")
        base = wid * per_w

        pltpu.sync_copy(idx_hbm.at[pl.ds(base, per_w)], idx_all)

        def start_gather(ci, rows, gsem):
            pltpu.async_copy(
                table_hbm.at[idx_all.at[pl.ds(ci * CHUNK, CHUNK)]], rows, gsem)

        def start_writeback(ci, rows, osem):
            pltpu.async_copy(rows, out_hbm.at[pl.ds(base + ci * CHUNK, CHUNK)],
                             osem)

        def wait_rows(rows, sem):
            pltpu.make_async_copy(table_hbm.at[pl.ds(0, CHUNK)], rows,
                                  sem).wait()

        start_gather(0, rows0, gsem0)

        def body(g, carry):
            ci0 = 2 * g

            wait_rows(rows0, gsem0)

            @pl.when(g > 0)
            def _():
                wait_rows(rows1, osem1)

            start_gather(ci0 + 1, rows1, gsem1)
            start_writeback(ci0, rows0, osem0)

            wait_rows(rows1, gsem1)

            @pl.when(g < n_groups - 1)
            def _():
                wait_rows(rows0, osem0)
                start_gather(ci0 + 2, rows0, gsem0)

            start_writeback(ci0 + 1, rows1, osem1)
            return carry

        lax.fori_loop(0, n_groups, body, 0)

        wait_rows(rows0, osem0)
        wait_rows(rows1, osem1)

    return gather_kernel


def kernel(batch, table):
    idx = batch.reshape(N).astype(jnp.int32)
    out = _build(N, D)(table, idx)
    return out.reshape(B, L, D)
